# Initial kernel scaffold; baseline (speedup 1.0000x reference)
import jax
import jax.numpy as jnp
from jax.experimental import pallas as pl


def kernel(features, coords, W1, W2):
    raise NotImplementedError("write your pallas kernel here")


# speedup vs baseline: 200.7713x; 200.7713x over previous
import functools

import jax
import jax.numpy as jnp
from jax import lax
from jax.experimental import pallas as pl
from jax.experimental.pallas import tpu as pltpu
from jax.experimental.pallas import tpu_sc as plsc

N = 100000
GRID_D = 128
LINSIZE = GRID_D * GRID_D * GRID_D
SENT = LINSIZE
DUMP = LINSIZE + 8
NW = 32
CPW = 3200
NPAD = NW * CPW
PADPTS = NPAD - N
CH = NPAD + 16
G_PER_W = 17 * 8192
G = 16 * G_PER_W

OFFS = [(dz, dy, dx) for dz in (-1, 0, 1) for dy in (-1, 0, 1)
        for dx in (-1, 0, 1)]
DK = [dz * (GRID_D * GRID_D) + dy * GRID_D + dx for dz, dy, dx in OFFS]

_mesh1 = plsc.VectorSubcoreMesh(core_axis_name="c", subcore_axis_name="s",
                                num_cores=1)
_mesh2 = plsc.VectorSubcoreMesh(core_axis_name="c", subcore_axis_name="s",
                                num_cores=2)


def _wid():
    return lax.axis_index("s") * 2 + lax.axis_index("c")


@functools.partial(
    pl.kernel,
    out_type=jax.ShapeDtypeStruct((G,), jnp.int32),
    mesh=_mesh1,
    compiler_params=pltpu.CompilerParams(needs_layout_passes=False),
    scratch_types=[
        pltpu.VMEM((8192,), jnp.int32),
        pltpu.VMEM((50, 128), jnp.int32),
        pltpu.VMEM((50, 128), jnp.int32),
        pltpu.SemaphoreType.DMA,
    ],
)
def _k1_grid(lin1, vals1, grid_out, zbuf, lidx, lval, sem):
    w = lax.axis_index("s")
    zero16 = jnp.zeros((16,), jnp.int32)

    @pl.loop(0, 512)
    def _fill(i):
        zbuf[pl.ds(i * 16, 16)] = zero16

    descs = [
        pltpu.async_copy(zbuf, grid_out.at[pl.ds(w * G_PER_W + t * 8192,
                                                 8192)], sem)
        for t in range(17)
    ]
    for d in descs:
        d.wait()
    plsc.subcore_barrier()

    descs = [
        pltpu.async_copy(lin1.at[pl.ds(w * 6400 + j * 128, 128)],
                         lidx.at[j], sem)
        for j in range(50)
    ] + [
        pltpu.async_copy(vals1.at[pl.ds(w * 6400 + j * 128, 128)],
                         lval.at[j], sem)
        for j in range(50)
    ]
    for d in descs:
        d.wait()
    descs = [
        pltpu.async_copy(lval.at[j], grid_out.at[lidx.at[j]], sem)
        for j in range(50)
    ]
    for d in descs:
        d.wait()


@functools.partial(
    pl.kernel,
    out_type=jax.ShapeDtypeStruct((27 * NPAD,), jnp.int32),
    mesh=_mesh2,
    compiler_params=pltpu.CompilerParams(needs_layout_passes=False),
    scratch_types=[
        pltpu.VMEM((CPW,), jnp.int32),
        pltpu.VMEM((CPW,), jnp.int32),
        pltpu.VMEM((CPW,), jnp.int32),
        pltpu.VMEM((27 * CPW,), jnp.int32),
        pltpu.VMEM((CPW,), jnp.int32),
        pltpu.SemaphoreType.DMA,
    ],
)
def _k2_rule(grid, coords1, rule, zv, yv, xv, idxbuf, nbrbuf, sem):
    w = _wid()
    base = w * CPW
    pltpu.sync_copy(coords1.at[pl.ds(base, CPW)], zv)
    pltpu.sync_copy(coords1.at[pl.ds(NPAD + base, CPW)], yv)
    pltpu.sync_copy(coords1.at[pl.ds(2 * NPAD + base, CPW)], xv)

    @pl.loop(0, CPW // 16)
    def _compute(g):
        o = g * 16
        zg = zv[pl.ds(o, 16)]
        yg = yv[pl.ds(o, 16)]
        xg = xv[pl.ds(o, 16)]
        ling = (zg * GRID_D + yg) * GRID_D + xg
        mz = {-1: zg >= 1, 0: None, 1: zg <= GRID_D - 2}
        my = {-1: yg >= 1, 0: None, 1: yg <= GRID_D - 2}
        mx = {-1: xg >= 1, 0: None, 1: xg <= GRID_D - 2}
        for k, (dz, dy, dx) in enumerate(OFFS):
            m = None
            for mm in (mz[dz], my[dy], mx[dx]):
                if mm is not None:
                    m = mm if m is None else (m & mm)
            idx = ling + DK[k]
            if m is not None:
                idx = jnp.where(m, idx, SENT)
            idxbuf[pl.ds(k * CPW + o, 16)] = idx

    @pl.loop(0, 27)
    def _gather(k):
        descs = [
            pltpu.async_copy(
                grid.at[idxbuf.at[pl.ds(k * CPW + j * 128, 128)]],
                nbrbuf.at[pl.ds(j * 128, 128)], sem)
            for j in range(CPW // 128)
        ]
        for d in descs:
            d.wait()
        pltpu.sync_copy(nbrbuf, rule.at[pl.ds(k * NPAD + base, CPW)])


@functools.partial(
    pl.kernel,
    out_type=jax.ShapeDtypeStruct((3 * CH,), jnp.float32),
    mesh=_mesh2,
    compiler_params=pltpu.CompilerParams(needs_layout_passes=False),
    scratch_types=[
        pltpu.VMEM((CH,), jnp.float32),
        pltpu.VMEM((27 * 320,), jnp.int32),
        pltpu.VMEM((3 * CPW,), jnp.float32),
        pltpu.VMEM((184,), jnp.float32),
        pltpu.VMEM((16,), jnp.float32),
        pltpu.SemaphoreType.DMA,
    ],
)
def _k3_conv1(rule, fcs, w1f, h, fc, ridx, hacc, w1v, z16, sem):
    w = _wid()
    base = w * CPW
    pltpu.sync_copy(w1f, w1v)
    for c in range(2):
        pltpu.sync_copy(fcs.at[pl.ds(c * CH, CH)], fc)

        @pl.loop(0, CPW // 320)
        def _sub(sc):
            descs = [
                pltpu.async_copy(
                    rule.at[pl.ds(k * NPAD + base + sc * 320, 320)],
                    ridx.at[pl.ds(k * 320, 320)], sem)
                for k in range(27)
            ]
            for d in descs:
                d.wait()

            @pl.loop(0, 20)
            def _grp(g):
                o = sc * 320 + g * 16
                if c == 0:
                    accs = [None, None, None]
                else:
                    accs = [hacc[pl.ds(j * CPW + o, 16)] for j in range(3)]
                for k in range(27):
                    idx = ridx[pl.ds(k * 320 + g * 16, 16)]
                    v = plsc.load_gather(fc, [idx])
                    for j in range(3):
                        wv = w1v[pl.ds(k * 6 + c * 3 + j, 16)][0]
                        t = wv * v
                        accs[j] = t if accs[j] is None else accs[j] + t
                for j in range(3):
                    hacc[pl.ds(j * CPW + o, 16)] = accs[j]

    @pl.when(w == 0)
    def _zero_head():
        z16[...] = jnp.zeros((16,), jnp.float32)
        for j in range(3):
            pltpu.sync_copy(z16, h.at[pl.ds(j * CH, 16)])
            pltpu.sync_copy(z16, h.at[pl.ds(j * CH + CH - 16, 16)])
    for j in range(3):
        pltpu.sync_copy(hacc.at[pl.ds(j * CPW, CPW)],
                        h.at[pl.ds(j * CH + 8 + base, CPW)])


@functools.partial(
    pl.kernel,
    out_type=jax.ShapeDtypeStruct((4 * NPAD,), jnp.float32),
    mesh=_mesh2,
    compiler_params=pltpu.CompilerParams(needs_layout_passes=False),
    scratch_types=[
        pltpu.VMEM((CH,), jnp.float32),
        pltpu.VMEM((27 * 320,), jnp.int32),
        pltpu.VMEM((4 * CPW,), jnp.float32),
        pltpu.VMEM((344,), jnp.float32),
        pltpu.SemaphoreType.DMA,
    ],
)
def _k4_conv2(rule, h, w2f, out, hch, ridx, oacc, w2v, sem):
    w = _wid()
    base = w * CPW
    pltpu.sync_copy(w2f, w2v)
    for c in range(3):
        pltpu.sync_copy(h.at[pl.ds(c * CH, CH)], hch)

        @pl.loop(0, CPW // 320)
        def _sub(sc):
            descs = [
                pltpu.async_copy(
                    rule.at[pl.ds(k * NPAD + base + sc * 320, 320)],
                    ridx.at[pl.ds(k * 320, 320)], sem)
                for k in range(27)
            ]
            for d in descs:
                d.wait()

            @pl.loop(0, 20)
            def _grp(g):
                o = sc * 320 + g * 16
                if c == 0:
                    accs = [None] * 4
                else:
                    accs = [oacc[pl.ds(j * CPW + o, 16)] for j in range(4)]
                for k in range(27):
                    idx = ridx[pl.ds(k * 320 + g * 16, 16)]
                    v = plsc.load_gather(hch, [idx])
                    for j in range(4):
                        wv = w2v[pl.ds(k * 12 + c * 4 + j, 16)][0]
                        t = wv * v
                        accs[j] = t if accs[j] is None else accs[j] + t
                for j in range(4):
                    oacc[pl.ds(j * CPW + o, 16)] = accs[j]

    for j in range(4):
        pltpu.sync_copy(oacc.at[pl.ds(j * CPW, CPW)],
                        out.at[pl.ds(j * NPAD + base, CPW)])


def kernel(features, coords, W1, W2):
    z = coords[:, 0].astype(jnp.int32)
    y = coords[:, 1].astype(jnp.int32)
    x = coords[:, 2].astype(jnp.int32)
    lin = (z * GRID_D + y) * GRID_D + x
    lin1 = jnp.concatenate(
        [lin, DUMP + jnp.arange(PADPTS, dtype=jnp.int32)])
    vals1 = jnp.arange(NPAD, dtype=jnp.int32) + 8

    pad1 = jnp.zeros((NPAD - N,), jnp.int32)
    coords1 = jnp.concatenate([z, pad1, y, pad1, x, pad1])

    fcs = jnp.zeros((2, CH), jnp.float32).at[:, 8:8 + N].set(
        features.T).reshape(-1)
    w1f = jnp.pad(W1.reshape(-1), (0, 184 - 162))
    w2f = jnp.pad(W2.reshape(-1), (0, 344 - 324))

    grid = _k1_grid(lin1, vals1)
    rule = _k2_rule(grid, coords1)
    h = _k3_conv1(rule, fcs, w1f)
    outT = _k4_conv2(rule, h, w2f)
    return outT.reshape(4, NPAD).T[:N]


# speedup vs baseline: 242.6555x; 1.2086x over previous
import functools

import jax
import jax.numpy as jnp
from jax import lax
from jax.experimental import pallas as pl
from jax.experimental.pallas import tpu as pltpu
from jax.experimental.pallas import tpu_sc as plsc

N = 100000
GRID_D = 128
NW = 32
CPW = 3200
NPAD = NW * CPW
PADPTS = NPAD - N
CH = NPAD + 16

AW = GRID_D * GRID_D * GRID_D
AROWS = AW // 16
BPITCH = 144
BW = GRID_D * GRID_D * BPITCH
SENTW = AW + BW
SROW = SENTW // 16
DUMPA = SENTW + 16
DUMPB = DUMPA + PADPTS
G_PER_W = 35 * 8192
G = 16 * G_PER_W
TROWS = G // 16

SB = 320
NSB = CPW // SB
RB = 9 * SB
RBP = 23 * 128

OFFS = [(dz, dy, dx) for dz in (-1, 0, 1) for dy in (-1, 0, 1)
        for dx in (-1, 0, 1)]
DD = [dz * GRID_D + dy for dz in (-1, 0, 1) for dy in (-1, 0, 1)]

_mesh1 = plsc.VectorSubcoreMesh(core_axis_name="c", subcore_axis_name="s",
                                num_cores=1)
_mesh2 = plsc.VectorSubcoreMesh(core_axis_name="c", subcore_axis_name="s",
                                num_cores=2)
_params = pltpu.CompilerParams(needs_layout_passes=False,
                               use_tc_tiling_on_sc=False)


def _wid():
    return lax.axis_index("s") * 2 + lax.axis_index("c")


@functools.partial(
    pl.kernel,
    out_type=jax.ShapeDtypeStruct((G,), jnp.int32),
    mesh=_mesh1,
    compiler_params=_params,
    scratch_types=[
        pltpu.VMEM((8192,), jnp.int32),
        pltpu.VMEM((100, 128), jnp.int32),
        pltpu.VMEM((50, 128), jnp.int32),
        pltpu.SemaphoreType.DMA,
    ],
)
def _k1_grid(lin1, vals1, grid_out, zbuf, lidx, lval, sem):
    w = lax.axis_index("s")
    zero16 = jnp.zeros((16,), jnp.int32)

    @pl.loop(0, 512)
    def _fill(i):
        zbuf[pl.ds(i * 16, 16)] = zero16

    descs = [
        pltpu.async_copy(zbuf, grid_out.at[pl.ds(w * G_PER_W + t * 8192,
                                                 8192)], sem)
        for t in range(35)
    ]
    for d in descs:
        d.wait()
    plsc.subcore_barrier()

    descs = [
        pltpu.async_copy(lin1.at[pl.ds(w * 6400 + j * 128, 128)],
                         lidx.at[j], sem)
        for j in range(50)
    ] + [
        pltpu.async_copy(lin1.at[pl.ds(NPAD + w * 6400 + j * 128, 128)],
                         lidx.at[50 + j], sem)
        for j in range(50)
    ] + [
        pltpu.async_copy(vals1.at[pl.ds(w * 6400 + j * 128, 128)],
                         lval.at[j], sem)
        for j in range(50)
    ]
    for d in descs:
        d.wait()
    descs = [
        pltpu.async_copy(lval.at[j], grid_out.at[lidx.at[j]], sem)
        for j in range(50)
    ] + [
        pltpu.async_copy(lval.at[j], grid_out.at[lidx.at[50 + j]], sem)
        for j in range(50)
    ]
    for d in descs:
        d.wait()


@functools.partial(
    pl.kernel,
    out_type=jax.ShapeDtypeStruct((27 * NPAD,), jnp.int32),
    mesh=_mesh2,
    compiler_params=_params,
    scratch_types=[
        pltpu.VMEM((CPW,), jnp.int32),
        pltpu.VMEM((CPW,), jnp.int32),
        pltpu.VMEM((CPW,), jnp.int32),
        pltpu.VMEM((CPW,), jnp.int32),
        pltpu.VMEM((RBP,), jnp.int32),
        pltpu.VMEM((RBP, 16), jnp.int32),
        pltpu.VMEM((27 * SB,), jnp.int32),
        pltpu.SemaphoreType.DMA,
    ],
)
def _k2_rule(grid2, coords1, rule, zv, yv, xv, colbuf, rowbuf, tilebuf,
             rulebuf, sem):
    w = _wid()
    base = w * CPW
    pltpu.sync_copy(coords1.at[pl.ds(base, CPW)], zv)
    pltpu.sync_copy(coords1.at[pl.ds(NPAD + base, CPW)], yv)
    pltpu.sync_copy(coords1.at[pl.ds(2 * NPAD + base, CPW)], xv)

    srow16 = jnp.full((16,), SROW, jnp.int32)
    for i in range(RB // 16, RBP // 16):
        rowbuf[pl.ds(i * 16, 16)] = srow16

    for sb in range(NSB):

        @pl.loop(0, SB // 16)
        def _rows(g):
            o = sb * SB + g * 16
            zg = zv[pl.ds(o, 16)]
            yg = yv[pl.ds(o, 16)]
            xg = xv[pl.ds(o, 16)]
            xm = xg & 15
            is_b = (xm == 0) | (xm == 15)
            colbuf[pl.ds(o, 16)] = jnp.where(is_b, (xg + 8) & 15, xm)
            pitch = jnp.where(is_b, 9, 8)
            crow = jnp.where(is_b, AROWS + ((xg + 8) >> 4), xg >> 4)
            lineidx = zg * GRID_D + yg
            mz = {-1: zg >= 1, 0: None, 1: zg <= GRID_D - 2}
            my = {-1: yg >= 1, 0: None, 1: yg <= GRID_D - 2}
            for dzy, (dz, dy) in enumerate(
                    (dz, dy) for dz in (-1, 0, 1) for dy in (-1, 0, 1)):
                row = (lineidx + DD[dzy]) * pitch + crow
                m = None
                for mm in (mz[dz], my[dy]):
                    if mm is not None:
                        m = mm if m is None else (m & mm)
                if m is not None:
                    row = jnp.where(m, row, SROW)
                rowbuf[pl.ds(dzy * SB + g * 16, 16)] = row

        descs = [
            pltpu.async_copy(grid2.at[rowbuf.at[pl.ds(j * 128, 128)]],
                             tilebuf.at[pl.ds(j * 128, 128)], sem)
            for j in range(RBP // 128)
        ]
        for d in descs:
            d.wait()

        @pl.loop(0, SB // 16)
        def _extract(g):
            colv = colbuf[pl.ds(sb * SB + g * 16, 16)]
            cols = [colv - 1, colv, colv + 1]
            rbg = jnp.arange(16, dtype=jnp.int32) + g * 16
            rows = [rbg + dzy * SB for dzy in range(9)]
            for k in range(27):
                v = plsc.load_gather(tilebuf, [rows[k // 3], cols[k % 3]])
                rulebuf[pl.ds(k * SB + g * 16, 16)] = v

        descs = [
            pltpu.async_copy(
                rulebuf.at[pl.ds(k * SB, SB)],
                rule.at[pl.ds(k * NPAD + base + sb * SB, SB)], sem)
            for k in range(27)
        ]
        for d in descs:
            d.wait()


@functools.partial(
    pl.kernel,
    out_type=jax.ShapeDtypeStruct((3 * CH,), jnp.float32),
    mesh=_mesh2,
    compiler_params=_params,
    scratch_types=[
        pltpu.VMEM((CH,), jnp.float32),
        pltpu.VMEM((27 * 320,), jnp.int32),
        pltpu.VMEM((3 * CPW,), jnp.float32),
        pltpu.VMEM((184,), jnp.float32),
        pltpu.VMEM((16,), jnp.float32),
        pltpu.SemaphoreType.DMA,
    ],
)
def _k3_conv1(rule, fcs, w1f, h, fc, ridx, hacc, w1v, z16, sem):
    w = _wid()
    base = w * CPW
    pltpu.sync_copy(w1f, w1v)
    for c in range(2):
        pltpu.sync_copy(fcs.at[pl.ds(c * CH, CH)], fc)

        @pl.loop(0, CPW // 320)
        def _sub(sc):
            descs = [
                pltpu.async_copy(
                    rule.at[pl.ds(k * NPAD + base + sc * 320, 320)],
                    ridx.at[pl.ds(k * 320, 320)], sem)
                for k in range(27)
            ]
            for d in descs:
                d.wait()

            @pl.loop(0, 20)
            def _grp(g):
                o = sc * 320 + g * 16
                if c == 0:
                    accs = [None, None, None]
                else:
                    accs = [hacc[pl.ds(j * CPW + o, 16)] for j in range(3)]
                for k in range(27):
                    idx = ridx[pl.ds(k * 320 + g * 16, 16)]
                    v = plsc.load_gather(fc, [idx])
                    for j in range(3):
                        wv = w1v[pl.ds(k * 6 + c * 3 + j, 16)][0]
                        t = wv * v
                        accs[j] = t if accs[j] is None else accs[j] + t
                for j in range(3):
                    hacc[pl.ds(j * CPW + o, 16)] = accs[j]

    @pl.when(w == 0)
    def _zero_head():
        z16[...] = jnp.zeros((16,), jnp.float32)
        for j in range(3):
            pltpu.sync_copy(z16, h.at[pl.ds(j * CH, 16)])
            pltpu.sync_copy(z16, h.at[pl.ds(j * CH + CH - 16, 16)])
    for j in range(3):
        pltpu.sync_copy(hacc.at[pl.ds(j * CPW, CPW)],
                        h.at[pl.ds(j * CH + 8 + base, CPW)])


@functools.partial(
    pl.kernel,
    out_type=jax.ShapeDtypeStruct((4 * NPAD,), jnp.float32),
    mesh=_mesh2,
    compiler_params=_params,
    scratch_types=[
        pltpu.VMEM((CH,), jnp.float32),
        pltpu.VMEM((27 * 320,), jnp.int32),
        pltpu.VMEM((4 * CPW,), jnp.float32),
        pltpu.VMEM((344,), jnp.float32),
        pltpu.SemaphoreType.DMA,
    ],
)
def _k4_conv2(rule, h, w2f, out, hch, ridx, oacc, w2v, sem):
    w = _wid()
    base = w * CPW
    pltpu.sync_copy(w2f, w2v)
    for c in range(3):
        pltpu.sync_copy(h.at[pl.ds(c * CH, CH)], hch)

        @pl.loop(0, CPW // 320)
        def _sub(sc):
            descs = [
                pltpu.async_copy(
                    rule.at[pl.ds(k * NPAD + base + sc * 320, 320)],
                    ridx.at[pl.ds(k * 320, 320)], sem)
                for k in range(27)
            ]
            for d in descs:
                d.wait()

            @pl.loop(0, 20)
            def _grp(g):
                o = sc * 320 + g * 16
                if c == 0:
                    accs = [None] * 4
                else:
                    accs = [oacc[pl.ds(j * CPW + o, 16)] for j in range(4)]
                for k in range(27):
                    idx = ridx[pl.ds(k * 320 + g * 16, 16)]
                    v = plsc.load_gather(hch, [idx])
                    for j in range(4):
                        wv = w2v[pl.ds(k * 12 + c * 4 + j, 16)][0]
                        t = wv * v
                        accs[j] = t if accs[j] is None else accs[j] + t
                for j in range(4):
                    oacc[pl.ds(j * CPW + o, 16)] = accs[j]

    for j in range(4):
        pltpu.sync_copy(oacc.at[pl.ds(j * CPW, CPW)],
                        out.at[pl.ds(j * NPAD + base, CPW)])


def kernel(features, coords, W1, W2):
    z = coords[:, 0].astype(jnp.int32)
    y = coords[:, 1].astype(jnp.int32)
    x = coords[:, 2].astype(jnp.int32)
    line = z * GRID_D + y
    lin_a = line * GRID_D + x
    lin_b = AW + line * BPITCH + (x + 8)
    dump = jnp.arange(PADPTS, dtype=jnp.int32)
    lin1 = jnp.concatenate([lin_a, DUMPA + dump, lin_b, DUMPB + dump])
    vals1 = jnp.arange(NPAD, dtype=jnp.int32) + 8

    pad1 = jnp.zeros((NPAD - N,), jnp.int32)
    coords1 = jnp.concatenate([z, pad1, y, pad1, x, pad1])

    fcs = jnp.zeros((2, CH), jnp.float32).at[:, 8:8 + N].set(
        features.T).reshape(-1)
    w1f = jnp.pad(W1.reshape(-1), (0, 184 - 162))
    w2f = jnp.pad(W2.reshape(-1), (0, 344 - 324))

    grid = _k1_grid(lin1, vals1)
    rule = _k2_rule(grid.reshape(TROWS, 16), coords1)
    h = _k3_conv1(rule, fcs, w1f)
    outT = _k4_conv2(rule, h, w2f)
    return outT.reshape(4, NPAD).T[:N]


# speedup vs baseline: 272.8084x; 1.1243x over previous
import functools

import jax
import jax.numpy as jnp
from jax import lax
from jax.experimental import pallas as pl
from jax.experimental.pallas import tpu as pltpu
from jax.experimental.pallas import tpu_sc as plsc

N = 100000
GRID_D = 128
NW = 32
CPW = 3200
NPAD = NW * CPW
PADPTS = NPAD - N
CH = NPAD + 16

AW = GRID_D * GRID_D * GRID_D
SROW = AW // 16
DUMPA = AW + 16
BPITCH = 144
BW = GRID_D * GRID_D * BPITCH
HALF = 16 * 19 * 8192
BOFF = HALF
BROW0 = BOFF // 16
DUMPB = BOFF + BW
G = 2 * HALF
TROWS = G // 16

SB = 320
NSB = CPW // SB
RB = 9 * SB
RBP = 23 * 128

OFFS = [(dz, dy, dx) for dz in (-1, 0, 1) for dy in (-1, 0, 1)
        for dx in (-1, 0, 1)]
DD = [dz * GRID_D + dy for dz in (-1, 0, 1) for dy in (-1, 0, 1)]

_mesh2 = plsc.VectorSubcoreMesh(core_axis_name="c", subcore_axis_name="s",
                                num_cores=2)
_params = pltpu.CompilerParams(needs_layout_passes=False,
                               use_tc_tiling_on_sc=False)


def _wid():
    return lax.axis_index("s") * 2 + lax.axis_index("c")


@functools.partial(
    pl.kernel,
    out_type=jax.ShapeDtypeStruct((G,), jnp.int32),
    mesh=_mesh2,
    compiler_params=_params,
    scratch_types=[
        pltpu.VMEM((8192,), jnp.int32),
        pltpu.VMEM((50, 128), jnp.int32),
        pltpu.VMEM((50, 128), jnp.int32),
        pltpu.SemaphoreType.DMA,
    ],
)
def _k1_grid(lin1, vals1, grid_out, zbuf, lidx, lval, sem):
    cid = lax.axis_index("c")
    w16 = lax.axis_index("s")
    zero16 = jnp.zeros((16,), jnp.int32)

    @pl.loop(0, 512)
    def _fill(i):
        zbuf[pl.ds(i * 16, 16)] = zero16

    mbase = cid * HALF + w16 * (19 * 8192)
    descs = [
        pltpu.async_copy(zbuf, grid_out.at[pl.ds(mbase + t * 8192, 8192)],
                         sem)
        for t in range(19)
    ]
    for d in descs:
        d.wait()
    plsc.subcore_barrier()

    lbase = cid * NPAD + w16 * 6400
    descs = [
        pltpu.async_copy(lin1.at[pl.ds(lbase + j * 128, 128)],
                         lidx.at[j], sem)
        for j in range(50)
    ] + [
        pltpu.async_copy(vals1.at[pl.ds(w16 * 6400 + j * 128, 128)],
                         lval.at[j], sem)
        for j in range(50)
    ]
    for d in descs:
        d.wait()
    descs = [
        pltpu.async_copy(lval.at[j], grid_out.at[lidx.at[j]], sem)
        for j in range(50)
    ]
    for d in descs:
        d.wait()


@functools.partial(
    pl.kernel,
    out_type=jax.ShapeDtypeStruct((27 * NPAD,), jnp.int32),
    mesh=_mesh2,
    compiler_params=_params,
    scratch_types=[
        pltpu.VMEM((CPW,), jnp.int32),
        pltpu.VMEM((CPW,), jnp.int32),
        pltpu.VMEM((CPW,), jnp.int32),
        pltpu.VMEM((CPW,), jnp.int32),
        pltpu.VMEM((2 * RBP,), jnp.int32),
        pltpu.VMEM((2 * RBP, 16), jnp.int32),
        pltpu.VMEM((27 * SB,), jnp.int32),
        pltpu.SemaphoreType.DMA,
        pltpu.SemaphoreType.DMA,
        pltpu.SemaphoreType.DMA,
    ],
)
def _k2_rule(grid2, coords1, rule, zv, yv, xv, colbuf, rowbuf, tilebuf,
             rulebuf, sg0, sg1, sw):
    w = _wid()
    base = w * CPW
    pltpu.sync_copy(coords1.at[pl.ds(base, CPW)], zv)
    pltpu.sync_copy(coords1.at[pl.ds(NPAD + base, CPW)], yv)
    pltpu.sync_copy(coords1.at[pl.ds(2 * NPAD + base, CPW)], xv)

    srow16 = jnp.full((16,), SROW, jnp.int32)
    for half in range(2):
        for i in range(RB // 16, RBP // 16):
            rowbuf[pl.ds(half * RBP + i * 16, 16)] = srow16

    def rowcompute(sb, half):
        @pl.loop(0, SB // 16)
        def _rows(g):
            o = sb * SB + g * 16
            zg = zv[pl.ds(o, 16)]
            yg = yv[pl.ds(o, 16)]
            xg = xv[pl.ds(o, 16)]
            xm = xg & 15
            is_b = (xm == 0) | (xm == 15)
            colbuf[pl.ds(o, 16)] = jnp.where(is_b, (xg + 8) & 15, xm)
            pitch = jnp.where(is_b, 9, 8)
            crow = jnp.where(is_b, BROW0 + ((xg + 8) >> 4), xg >> 4)
            lineidx = zg * GRID_D + yg
            mz = {-1: zg >= 1, 0: None, 1: zg <= GRID_D - 2}
            my = {-1: yg >= 1, 0: None, 1: yg <= GRID_D - 2}
            for dzy, (dz, dy) in enumerate(
                    (dz, dy) for dz in (-1, 0, 1) for dy in (-1, 0, 1)):
                row = (lineidx + DD[dzy]) * pitch + crow
                m = None
                for mm in (mz[dz], my[dy]):
                    if mm is not None:
                        m = mm if m is None else (m & mm)
                if m is not None:
                    row = jnp.where(m, row, SROW)
                rowbuf[pl.ds(half * RBP + dzy * SB + g * 16, 16)] = row

    def fire_gather(half, sg):
        for j in range(RBP // 128):
            pltpu.async_copy(
                grid2.at[rowbuf.at[pl.ds(half * RBP + j * 128, 128)]],
                tilebuf.at[pl.ds(half * RBP + j * 128, 128)], sg)

    def drain_gather(sg):
        pltpu.make_async_copy(grid2.at[pl.ds(0, RBP)],
                              tilebuf.at[pl.ds(0, RBP)], sg).wait()

    def extract_write(sb, half):
        @pl.loop(0, SB // 16)
        def _extract(g):
            colv = colbuf[pl.ds(sb * SB + g * 16, 16)]
            cols = [colv - 1, colv, colv + 1]
            rbg = jnp.arange(16, dtype=jnp.int32) + (half * RBP + g * 16)
            rows = [rbg + dzy * SB for dzy in range(9)]
            for k in range(27):
                v = plsc.load_gather(tilebuf, [rows[k // 3], cols[k % 3]])
                rulebuf[pl.ds(k * SB + g * 16, 16)] = v

        descs = [
            pltpu.async_copy(
                rulebuf.at[pl.ds(k * SB, SB)],
                rule.at[pl.ds(k * NPAD + base + sb * SB, SB)], sw)
            for k in range(27)
        ]
        for d in descs:
            d.wait()

    rowcompute(0, 0)
    fire_gather(0, sg0)

    @pl.loop(0, NSB // 2)
    def _pipe(i):
        sb0 = i * 2
        rowcompute(sb0 + 1, 1)
        fire_gather(1, sg1)
        drain_gather(sg0)
        extract_write(sb0, 0)

        @pl.when(sb0 + 2 < NSB)
        def _next():
            rowcompute(sb0 + 2, 0)
            fire_gather(0, sg0)

        drain_gather(sg1)
        extract_write(sb0 + 1, 1)


def _conv_body(rule, base, chans_in, chans_out, stage_chan, ridx, acc_buf,
               wv, subsz, s0, s1):
    nsub = CPW // subsz
    batch = 27 * subsz

    def fire(sc, boff, sem):
        for k in range(27):
            pltpu.async_copy(
                rule.at[pl.ds(k * NPAD + base + sc * subsz, subsz)],
                ridx.at[pl.ds(boff + k * subsz, subsz)], sem)

    def drain(sem):
        pltpu.make_async_copy(rule.at[pl.ds(0, batch)],
                              ridx.at[pl.ds(0, batch)], sem).wait()

    for c in range(chans_in):
        ch_ref = stage_chan(c)
        fire(0, 0, s0)

        @pl.loop(0, nsub // 2)
        def _pipe(i):
            sc0 = i * 2
            fire(sc0 + 1, batch, s1)
            drain(s0)
            _conv_compute(c, sc0, 0, ch_ref, ridx, acc_buf, wv, subsz,
                          chans_in, chans_out)

            @pl.when(sc0 + 2 < nsub)
            def _next():
                fire(sc0 + 2, 0, s0)

            drain(s1)
            _conv_compute(c, sc0 + 1, batch, ch_ref, ridx, acc_buf, wv,
                          subsz, chans_in, chans_out)


def _conv_compute(c, sc, boff, ch_ref, ridx, acc_buf, wv, subsz, chans_in,
                  chans_out):
    @pl.loop(0, subsz // 16)
    def _grp(g):
        o = sc * subsz + g * 16
        if c == 0:
            accs = [None] * chans_out
        else:
            accs = [acc_buf[pl.ds(j * CPW + o, 16)]
                    for j in range(chans_out)]
        for k in range(27):
            idx = ridx[pl.ds(boff + k * subsz + g * 16, 16)]
            v = plsc.load_gather(ch_ref, [idx])
            for j in range(chans_out):
                wvec = wv[pl.ds((k * chans_in + c) * chans_out + j, 16)][0]
                t = wvec * v
                accs[j] = t if accs[j] is None else accs[j] + t
        for j in range(chans_out):
            acc_buf[pl.ds(j * CPW + o, 16)] = accs[j]


@functools.partial(
    pl.kernel,
    out_type=jax.ShapeDtypeStruct((3 * CH,), jnp.float32),
    mesh=_mesh2,
    compiler_params=_params,
    scratch_types=[
        pltpu.VMEM((CH,), jnp.float32),
        pltpu.VMEM((2 * 27 * 320,), jnp.int32),
        pltpu.VMEM((3 * CPW,), jnp.float32),
        pltpu.VMEM((184,), jnp.float32),
        pltpu.VMEM((16,), jnp.float32),
        pltpu.SemaphoreType.DMA,
        pltpu.SemaphoreType.DMA,
    ],
)
def _k3_conv1(rule, fcs, w1f, h, fc, ridx, hacc, w1v, z16, s0, s1):
    w = _wid()
    base = w * CPW
    pltpu.sync_copy(w1f, w1v)

    def stage_chan(c):
        pltpu.sync_copy(fcs.at[pl.ds(c * CH, CH)], fc)
        return fc

    _conv_body(rule, base, 2, 3, stage_chan, ridx, hacc, w1v, 320, s0, s1)

    @pl.when(w == 0)
    def _zero_head():
        z16[...] = jnp.zeros((16,), jnp.float32)
        for j in range(3):
            pltpu.sync_copy(z16, h.at[pl.ds(j * CH, 16)])
            pltpu.sync_copy(z16, h.at[pl.ds(j * CH + CH - 16, 16)])
    for j in range(3):
        pltpu.sync_copy(hacc.at[pl.ds(j * CPW, CPW)],
                        h.at[pl.ds(j * CH + 8 + base, CPW)])


@functools.partial(
    pl.kernel,
    out_type=jax.ShapeDtypeStruct((4 * NPAD,), jnp.float32),
    mesh=_mesh2,
    compiler_params=_params,
    scratch_types=[
        pltpu.VMEM((CH,), jnp.float32),
        pltpu.VMEM((2 * 27 * 160,), jnp.int32),
        pltpu.VMEM((4 * CPW,), jnp.float32),
        pltpu.VMEM((344,), jnp.float32),
        pltpu.SemaphoreType.DMA,
        pltpu.SemaphoreType.DMA,
    ],
)
def _k4_conv2(rule, h, w2f, out, hch, ridx, oacc, w2v, s0, s1):
    w = _wid()
    base = w * CPW
    pltpu.sync_copy(w2f, w2v)

    def stage_chan(c):
        pltpu.sync_copy(h.at[pl.ds(c * CH, CH)], hch)
        return hch

    _conv_body(rule, base, 3, 4, stage_chan, ridx, oacc, w2v, 160, s0, s1)

    for j in range(4):
        pltpu.sync_copy(oacc.at[pl.ds(j * CPW, CPW)],
                        out.at[pl.ds(j * NPAD + base, CPW)])


def kernel(features, coords, W1, W2):
    z = coords[:, 0].astype(jnp.int32)
    y = coords[:, 1].astype(jnp.int32)
    x = coords[:, 2].astype(jnp.int32)
    line = z * GRID_D + y
    lin_a = line * GRID_D + x
    lin_b = BOFF + line * BPITCH + (x + 8)
    dump = jnp.arange(PADPTS, dtype=jnp.int32)
    lin1 = jnp.concatenate([lin_a, DUMPA + dump, lin_b, DUMPB + dump])
    vals1 = jnp.arange(NPAD, dtype=jnp.int32) + 8

    pad1 = jnp.zeros((NPAD - N,), jnp.int32)
    coords1 = jnp.concatenate([z, pad1, y, pad1, x, pad1])

    fcs = jnp.zeros((2, CH), jnp.float32).at[:, 8:8 + N].set(
        features.T).reshape(-1)
    w1f = jnp.pad(W1.reshape(-1), (0, 184 - 162))
    w2f = jnp.pad(W2.reshape(-1), (0, 344 - 324))

    grid = _k1_grid(lin1, vals1)
    rule = _k2_rule(grid.reshape(TROWS, 16), coords1)
    h = _k3_conv1(rule, fcs, w1f)
    outT = _k4_conv2(rule, h, w2f)
    return outT.reshape(4, NPAD).T[:N]


# speedup vs baseline: 278.4264x; 1.0206x over previous
import functools

import jax
import jax.numpy as jnp
from jax import lax
from jax.experimental import pallas as pl
from jax.experimental.pallas import tpu as pltpu
from jax.experimental.pallas import tpu_sc as plsc

N = 100000
GRID_D = 128
NW = 32
CPW = 3200
NPAD = NW * CPW
PADPTS = NPAD - N
CH = NPAD + 16

AW = GRID_D * GRID_D * GRID_D
SROW = AW // 16
DUMPA = AW + 16
BPITCH = 144
BW = GRID_D * GRID_D * BPITCH
HALF = 16 * 19 * 8192
BOFF = HALF
BROW0 = BOFF // 16
DUMPB = BOFF + BW
G = 2 * HALF
TROWS = G // 16

SB = 320
NSB = CPW // SB
RB = 9 * SB
RBP = 23 * 128

OFFS = [(dz, dy, dx) for dz in (-1, 0, 1) for dy in (-1, 0, 1)
        for dx in (-1, 0, 1)]
DD = [dz * GRID_D + dy for dz in (-1, 0, 1) for dy in (-1, 0, 1)]

_mesh2 = plsc.VectorSubcoreMesh(core_axis_name="c", subcore_axis_name="s",
                                num_cores=2)
_params = pltpu.CompilerParams(needs_layout_passes=False,
                               use_tc_tiling_on_sc=False)


def _wid():
    return lax.axis_index("s") * 2 + lax.axis_index("c")


@functools.partial(
    pl.kernel,
    out_type=jax.ShapeDtypeStruct((G,), jnp.int32),
    mesh=_mesh2,
    compiler_params=_params,
    scratch_types=[
        pltpu.VMEM((8192,), jnp.int32),
        pltpu.VMEM((50, 128), jnp.int32),
        pltpu.VMEM((50, 128), jnp.int32),
        pltpu.SemaphoreType.DMA,
    ],
)
def _k1_grid(lin1, vals1, grid_out, zbuf, lidx, lval, sem):
    cid = lax.axis_index("c")
    w16 = lax.axis_index("s")
    zero16 = jnp.zeros((16,), jnp.int32)

    @pl.loop(0, 512)
    def _fill(i):
        zbuf[pl.ds(i * 16, 16)] = zero16

    mbase = cid * HALF + w16 * (19 * 8192)
    descs = [
        pltpu.async_copy(zbuf, grid_out.at[pl.ds(mbase + t * 8192, 8192)],
                         sem)
        for t in range(19)
    ]
    for d in descs:
        d.wait()
    plsc.subcore_barrier()

    lbase = cid * NPAD + w16 * 6400
    descs = [
        pltpu.async_copy(lin1.at[pl.ds(lbase + j * 128, 128)],
                         lidx.at[j], sem)
        for j in range(50)
    ] + [
        pltpu.async_copy(vals1.at[pl.ds(w16 * 6400 + j * 128, 128)],
                         lval.at[j], sem)
        for j in range(50)
    ]
    for d in descs:
        d.wait()
    descs = [
        pltpu.async_copy(lval.at[j], grid_out.at[lidx.at[j]], sem)
        for j in range(50)
    ]
    for d in descs:
        d.wait()


@functools.partial(
    pl.kernel,
    out_type=jax.ShapeDtypeStruct((27 * NPAD,), jnp.int32),
    mesh=_mesh2,
    compiler_params=_params,
    scratch_types=[
        pltpu.VMEM((CPW,), jnp.int32),
        pltpu.VMEM((CPW,), jnp.int32),
        pltpu.VMEM((CPW,), jnp.int32),
        pltpu.VMEM((CPW,), jnp.int32),
        pltpu.VMEM((2 * RBP,), jnp.int32),
        pltpu.VMEM((2 * RBP, 16), jnp.int32),
        pltpu.VMEM((27 * SB,), jnp.int32),
        pltpu.SemaphoreType.DMA,
        pltpu.SemaphoreType.DMA,
        pltpu.SemaphoreType.DMA,
    ],
)
def _k2_rule(grid2, coords1, rule, zv, yv, xv, colbuf, rowbuf, tilebuf,
             rulebuf, sg0, sg1, sw):
    w = _wid()
    base = w * CPW
    pltpu.sync_copy(coords1.at[pl.ds(base, CPW)], zv)
    pltpu.sync_copy(coords1.at[pl.ds(NPAD + base, CPW)], yv)
    pltpu.sync_copy(coords1.at[pl.ds(2 * NPAD + base, CPW)], xv)

    srow16 = jnp.full((16,), SROW, jnp.int32)
    for half in range(2):
        for i in range(RB // 16, RBP // 16):
            rowbuf[pl.ds(half * RBP + i * 16, 16)] = srow16

    def rowcompute(sb, half):
        @pl.loop(0, SB // 16)
        def _rows(g):
            o = sb * SB + g * 16
            zg = zv[pl.ds(o, 16)]
            yg = yv[pl.ds(o, 16)]
            xg = xv[pl.ds(o, 16)]
            xm = xg & 15
            is_b = (xm == 0) | (xm == 15)
            colbuf[pl.ds(o, 16)] = jnp.where(is_b, (xg + 8) & 15, xm)
            pitch = jnp.where(is_b, 9, 8)
            crow = jnp.where(is_b, BROW0 + ((xg + 8) >> 4), xg >> 4)
            lineidx = zg * GRID_D + yg
            mz = {-1: zg >= 1, 0: None, 1: zg <= GRID_D - 2}
            my = {-1: yg >= 1, 0: None, 1: yg <= GRID_D - 2}
            for dzy, (dz, dy) in enumerate(
                    (dz, dy) for dz in (-1, 0, 1) for dy in (-1, 0, 1)):
                row = (lineidx + DD[dzy]) * pitch + crow
                m = None
                for mm in (mz[dz], my[dy]):
                    if mm is not None:
                        m = mm if m is None else (m & mm)
                if m is not None:
                    row = jnp.where(m, row, SROW)
                rowbuf[pl.ds(half * RBP + dzy * SB + g * 16, 16)] = row

    def fire_gather(half, sg):
        for j in range(RBP // 128):
            pltpu.async_copy(
                grid2.at[rowbuf.at[pl.ds(half * RBP + j * 128, 128)]],
                tilebuf.at[pl.ds(half * RBP + j * 128, 128)], sg)

    def drain_gather(sg):
        pltpu.make_async_copy(grid2.at[pl.ds(0, RBP)],
                              tilebuf.at[pl.ds(0, RBP)], sg).wait()

    def extract_write(sb, half):
        @pl.loop(0, SB // 32)
        def _extract(gg):
            for gh in (0, 1):
                g = gg * 2 + gh
                colv = colbuf[pl.ds(sb * SB + g * 16, 16)]
                cols = [colv - 1, colv, colv + 1]
                rbg = jnp.arange(16, dtype=jnp.int32) + (half * RBP
                                                         + g * 16)
                rows = [rbg + dzy * SB for dzy in range(9)]
                for k in range(27):
                    v = plsc.load_gather(tilebuf,
                                         [rows[k // 3], cols[k % 3]])
                    rulebuf[pl.ds(k * SB + g * 16, 16)] = v

        descs = [
            pltpu.async_copy(
                rulebuf.at[pl.ds(k * SB, SB)],
                rule.at[pl.ds(k * NPAD + base + sb * SB, SB)], sw)
            for k in range(27)
        ]
        for d in descs:
            d.wait()

    rowcompute(0, 0)
    fire_gather(0, sg0)

    @pl.loop(0, NSB // 2)
    def _pipe(i):
        sb0 = i * 2
        rowcompute(sb0 + 1, 1)
        fire_gather(1, sg1)
        drain_gather(sg0)
        extract_write(sb0, 0)

        @pl.when(sb0 + 2 < NSB)
        def _next():
            rowcompute(sb0 + 2, 0)
            fire_gather(0, sg0)

        drain_gather(sg1)
        extract_write(sb0 + 1, 1)


def _conv_body(rule, base, chans_in, chans_out, stage_chan, ridx, acc_buf,
               wv, subsz, s0, s1):
    nsub = CPW // subsz
    batch = 27 * subsz

    def fire(sc, boff, sem):
        for k in range(27):
            pltpu.async_copy(
                rule.at[pl.ds(k * NPAD + base + sc * subsz, subsz)],
                ridx.at[pl.ds(boff + k * subsz, subsz)], sem)

    def drain(sem):
        pltpu.make_async_copy(rule.at[pl.ds(0, batch)],
                              ridx.at[pl.ds(0, batch)], sem).wait()

    for c in range(chans_in):
        ch_ref = stage_chan(c)
        fire(0, 0, s0)

        @pl.loop(0, nsub // 2)
        def _pipe(i):
            sc0 = i * 2
            fire(sc0 + 1, batch, s1)
            drain(s0)
            _conv_compute(c, sc0, 0, ch_ref, ridx, acc_buf, wv, subsz,
                          chans_in, chans_out)

            @pl.when(sc0 + 2 < nsub)
            def _next():
                fire(sc0 + 2, 0, s0)

            drain(s1)
            _conv_compute(c, sc0 + 1, batch, ch_ref, ridx, acc_buf, wv,
                          subsz, chans_in, chans_out)


def _conv_compute(c, sc, boff, ch_ref, ridx, acc_buf, wv, subsz, chans_in,
                  chans_out):
    @pl.loop(0, subsz // 32)
    def _grp(gg):
        for half in (0, 1):
            g = gg * 2 + half
            o = sc * subsz + g * 16
            if c == 0:
                accs = [None] * chans_out
            else:
                accs = [acc_buf[pl.ds(j * CPW + o, 16)]
                        for j in range(chans_out)]
            for k in range(27):
                idx = ridx[pl.ds(boff + k * subsz + g * 16, 16)]
                v = plsc.load_gather(ch_ref, [idx])
                for j in range(chans_out):
                    wvec = wv[pl.ds((k * chans_in + c) * chans_out + j,
                                    16)][0]
                    t = wvec * v
                    accs[j] = t if accs[j] is None else accs[j] + t
            for j in range(chans_out):
                acc_buf[pl.ds(j * CPW + o, 16)] = accs[j]


@functools.partial(
    pl.kernel,
    out_type=jax.ShapeDtypeStruct((3 * CH,), jnp.float32),
    mesh=_mesh2,
    compiler_params=_params,
    scratch_types=[
        pltpu.VMEM((CH,), jnp.float32),
        pltpu.VMEM((2 * 27 * 320,), jnp.int32),
        pltpu.VMEM((3 * CPW,), jnp.float32),
        pltpu.VMEM((184,), jnp.float32),
        pltpu.VMEM((16,), jnp.float32),
        pltpu.SemaphoreType.DMA,
        pltpu.SemaphoreType.DMA,
    ],
)
def _k3_conv1(rule, fcs, w1f, h, fc, ridx, hacc, w1v, z16, s0, s1):
    w = _wid()
    base = w * CPW
    pltpu.sync_copy(w1f, w1v)

    def stage_chan(c):
        pltpu.sync_copy(fcs.at[pl.ds(c * CH, CH)], fc)
        return fc

    _conv_body(rule, base, 2, 3, stage_chan, ridx, hacc, w1v, 320, s0, s1)

    @pl.when(w == 0)
    def _zero_head():
        z16[...] = jnp.zeros((16,), jnp.float32)
        for j in range(3):
            pltpu.sync_copy(z16, h.at[pl.ds(j * CH, 16)])
            pltpu.sync_copy(z16, h.at[pl.ds(j * CH + CH - 16, 16)])
    for j in range(3):
        pltpu.sync_copy(hacc.at[pl.ds(j * CPW, CPW)],
                        h.at[pl.ds(j * CH + 8 + base, CPW)])


@functools.partial(
    pl.kernel,
    out_type=jax.ShapeDtypeStruct((4 * NPAD,), jnp.float32),
    mesh=_mesh2,
    compiler_params=_params,
    scratch_types=[
        pltpu.VMEM((CH,), jnp.float32),
        pltpu.VMEM((2 * 27 * 160,), jnp.int32),
        pltpu.VMEM((4 * CPW,), jnp.float32),
        pltpu.VMEM((344,), jnp.float32),
        pltpu.SemaphoreType.DMA,
        pltpu.SemaphoreType.DMA,
    ],
)
def _k4_conv2(rule, h, w2f, out, hch, ridx, oacc, w2v, s0, s1):
    w = _wid()
    base = w * CPW
    pltpu.sync_copy(w2f, w2v)

    def stage_chan(c):
        pltpu.sync_copy(h.at[pl.ds(c * CH, CH)], hch)
        return hch

    _conv_body(rule, base, 3, 4, stage_chan, ridx, oacc, w2v, 160, s0, s1)

    for j in range(4):
        pltpu.sync_copy(oacc.at[pl.ds(j * CPW, CPW)],
                        out.at[pl.ds(j * NPAD + base, CPW)])


def kernel(features, coords, W1, W2):
    z = coords[:, 0].astype(jnp.int32)
    y = coords[:, 1].astype(jnp.int32)
    x = coords[:, 2].astype(jnp.int32)
    line = z * GRID_D + y
    lin_a = line * GRID_D + x
    lin_b = BOFF + line * BPITCH + (x + 8)
    dump = jnp.arange(PADPTS, dtype=jnp.int32)
    lin1 = jnp.concatenate([lin_a, DUMPA + dump, lin_b, DUMPB + dump])
    vals1 = jnp.arange(NPAD, dtype=jnp.int32) + 8

    pad1 = jnp.zeros((NPAD - N,), jnp.int32)
    coords1 = jnp.concatenate([z, pad1, y, pad1, x, pad1])

    fcs = jnp.zeros((2, CH), jnp.float32).at[:, 8:8 + N].set(
        features.T).reshape(-1)
    w1f = jnp.pad(W1.reshape(-1), (0, 184 - 162))
    w2f = jnp.pad(W2.reshape(-1), (0, 344 - 324))

    grid = _k1_grid(lin1, vals1)
    rule = _k2_rule(grid.reshape(TROWS, 16), coords1)
    h = _k3_conv1(rule, fcs, w1f)
    outT = _k4_conv2(rule, h, w2f)
    return outT.reshape(4, NPAD).T[:N]


# speedup vs baseline: 296.7492x; 1.0658x over previous
import functools

import jax
import jax.numpy as jnp
from jax import lax
from jax.experimental import pallas as pl
from jax.experimental.pallas import tpu as pltpu
from jax.experimental.pallas import tpu_sc as plsc

N = 100000
GRID_D = 128
NW = 32
CPW = 3200
NPAD = NW * CPW
PADPTS = NPAD - N
CH = NPAD + 16

AW = GRID_D * GRID_D * GRID_D
SROW = AW // 16
DUMPA = AW + 16
BPITCH = 144
BW = GRID_D * GRID_D * BPITCH
HALF0 = 16 * 17 * 8192
BOFF = HALF0
BROW0 = BOFF // 16
G = HALF0 + BW
TROWS = G // 16

SB = 320
NSB = CPW // SB
RB = 9 * SB
RBP = 23 * 128

OFFS = [(dz, dy, dx) for dz in (-1, 0, 1) for dy in (-1, 0, 1)
        for dx in (-1, 0, 1)]
DD = [dz * GRID_D + dy for dz in (-1, 0, 1) for dy in (-1, 0, 1)]

_mesh2 = plsc.VectorSubcoreMesh(core_axis_name="c", subcore_axis_name="s",
                                num_cores=2)
_params = pltpu.CompilerParams(needs_layout_passes=False,
                               use_tc_tiling_on_sc=False)


def _wid():
    return lax.axis_index("s") * 2 + lax.axis_index("c")


@functools.partial(
    pl.kernel,
    out_type=jax.ShapeDtypeStruct((HALF0,), jnp.int32),
    mesh=_mesh2,
    compiler_params=_params,
    scratch_types=[
        pltpu.VMEM((8192,), jnp.int32),
        pltpu.VMEM((50, 128), jnp.int32),
        pltpu.VMEM((50, 128), jnp.int32),
        pltpu.SemaphoreType.DMA,
    ],
)
def _k1a_scatter(lin1, vals1, ga_out, zbuf, lidx, lval, sem):
    cid = lax.axis_index("c")
    w16 = lax.axis_index("s")

    @pl.when(cid == 0)
    def _work():
        zero16 = jnp.zeros((16,), jnp.int32)

        @pl.loop(0, 512)
        def _fill(i):
            zbuf[pl.ds(i * 16, 16)] = zero16

        mbase = w16 * (17 * 8192)
        descs = [
            pltpu.async_copy(zbuf, ga_out.at[pl.ds(mbase + t * 8192,
                                                   8192)], sem)
            for t in range(17)
        ]
        lbase = w16 * 6400
        descs += [
            pltpu.async_copy(lin1.at[pl.ds(lbase + j * 128, 128)],
                             lidx.at[j], sem)
            for j in range(50)
        ] + [
            pltpu.async_copy(vals1.at[pl.ds(lbase + j * 128, 128)],
                             lval.at[j], sem)
            for j in range(50)
        ]
        for d in descs:
            d.wait()
        plsc.subcore_barrier()

        descs = [
            pltpu.async_copy(lval.at[j], ga_out.at[lidx.at[j]], sem)
            for j in range(50)
        ]
        for d in descs:
            d.wait()


@functools.partial(
    pl.kernel,
    out_type=jax.ShapeDtypeStruct((G,), jnp.int32),
    mesh=_mesh2,
    compiler_params=_params,
    scratch_types=[
        pltpu.VMEM((8192,), jnp.int32),
        pltpu.VMEM((9232,), jnp.int32),
        pltpu.SemaphoreType.DMA,
    ],
)
def _k1b_mirror(ga, grid_out, abuf, bbuf, sem):
    w = _wid()
    zero16 = jnp.zeros((16,), jnp.int32)
    for ln in range(64):
        bbuf[pl.ds(ln * 144 + 136, 16)] = zero16
    bbuf[pl.ds(0, 16)] = zero16

    for t in range(8):
        pltpu.sync_copy(ga.at[pl.ds(w * 65536 + t * 8192, 8192)], abuf)
        d = pltpu.async_copy(
            abuf, grid_out.at[pl.ds(w * 65536 + t * 8192, 8192)], sem)

        @pl.loop(0, 64)
        def _compose(ln):
            for q in range(8):
                bbuf[pl.ds(ln * 144 + 8 + q * 16, 16)] = (
                    abuf[pl.ds(ln * 128 + q * 16, 16)])

        pltpu.sync_copy(
            bbuf.at[pl.ds(0, 9216)],
            grid_out.at[pl.ds(BOFF + w * 73728 + t * 9216, 9216)])
        d.wait()

    @pl.when(w == 0)
    def _sentinel():
        abuf[pl.ds(0, 16)] = zero16
        pltpu.sync_copy(abuf.at[pl.ds(0, 16)],
                        grid_out.at[pl.ds(AW, 16)])


@functools.partial(
    pl.kernel,
    out_type=jax.ShapeDtypeStruct((27 * NPAD,), jnp.int32),
    mesh=_mesh2,
    compiler_params=_params,
    scratch_types=[
        pltpu.VMEM((CPW,), jnp.int32),
        pltpu.VMEM((CPW,), jnp.int32),
        pltpu.VMEM((CPW,), jnp.int32),
        pltpu.VMEM((CPW,), jnp.int32),
        pltpu.VMEM((2 * RBP,), jnp.int32),
        pltpu.VMEM((2 * RBP, 16), jnp.int32),
        pltpu.VMEM((27 * SB,), jnp.int32),
        pltpu.SemaphoreType.DMA,
        pltpu.SemaphoreType.DMA,
        pltpu.SemaphoreType.DMA,
    ],
)
def _k2_rule(grid2, coords1, rule, zv, yv, xv, colbuf, rowbuf, tilebuf,
             rulebuf, sg0, sg1, sw):
    w = _wid()
    base = w * CPW
    pltpu.sync_copy(coords1.at[pl.ds(base, CPW)], zv)
    pltpu.sync_copy(coords1.at[pl.ds(NPAD + base, CPW)], yv)
    pltpu.sync_copy(coords1.at[pl.ds(2 * NPAD + base, CPW)], xv)

    srow16 = jnp.full((16,), SROW, jnp.int32)
    for half in range(2):
        for i in range(RB // 16, RBP // 16):
            rowbuf[pl.ds(half * RBP + i * 16, 16)] = srow16

    def rowcompute(sb, half):
        @pl.loop(0, SB // 16)
        def _rows(g):
            o = sb * SB + g * 16
            zg = zv[pl.ds(o, 16)]
            yg = yv[pl.ds(o, 16)]
            xg = xv[pl.ds(o, 16)]
            xm = xg & 15
            is_b = (xm == 0) | (xm == 15)
            colbuf[pl.ds(o, 16)] = jnp.where(is_b, (xg + 8) & 15, xm)
            pitch = jnp.where(is_b, 9, 8)
            crow = jnp.where(is_b, BROW0 + ((xg + 8) >> 4), xg >> 4)
            lineidx = zg * GRID_D + yg
            mz = {-1: zg >= 1, 0: None, 1: zg <= GRID_D - 2}
            my = {-1: yg >= 1, 0: None, 1: yg <= GRID_D - 2}
            for dzy, (dz, dy) in enumerate(
                    (dz, dy) for dz in (-1, 0, 1) for dy in (-1, 0, 1)):
                row = (lineidx + DD[dzy]) * pitch + crow
                m = None
                for mm in (mz[dz], my[dy]):
                    if mm is not None:
                        m = mm if m is None else (m & mm)
                if m is not None:
                    row = jnp.where(m, row, SROW)
                rowbuf[pl.ds(half * RBP + dzy * SB + g * 16, 16)] = row

    def fire_gather(half, sg):
        for j in range(RBP // 128):
            pltpu.async_copy(
                grid2.at[rowbuf.at[pl.ds(half * RBP + j * 128, 128)]],
                tilebuf.at[pl.ds(half * RBP + j * 128, 128)], sg)

    def drain_gather(sg):
        for j in range(RBP // 128):
            pltpu.make_async_copy(grid2.at[pl.ds(0, 128)],
                                  tilebuf.at[pl.ds(0, 128)], sg).wait()

    def extract_write(sb, half):
        @pl.loop(0, SB // 32)
        def _extract(gg):
            for gh in (0, 1):
                g = gg * 2 + gh
                colv = colbuf[pl.ds(sb * SB + g * 16, 16)]
                cols = [colv - 1, colv, colv + 1]
                rbg = jnp.arange(16, dtype=jnp.int32) + (half * RBP
                                                         + g * 16)
                rows = [rbg + dzy * SB for dzy in range(9)]
                for k in range(27):
                    v = plsc.load_gather(tilebuf,
                                         [rows[k // 3], cols[k % 3]])
                    rulebuf[pl.ds(k * SB + g * 16, 16)] = v

        descs = [
            pltpu.async_copy(
                rulebuf.at[pl.ds(k * SB, SB)],
                rule.at[pl.ds(k * NPAD + base + sb * SB, SB)], sw)
            for k in range(27)
        ]
        for d in descs:
            d.wait()

    rowcompute(0, 0)
    fire_gather(0, sg0)

    @pl.loop(0, NSB // 2)
    def _pipe(i):
        sb0 = i * 2
        rowcompute(sb0 + 1, 1)
        fire_gather(1, sg1)
        drain_gather(sg0)
        extract_write(sb0, 0)

        @pl.when(sb0 + 2 < NSB)
        def _next():
            rowcompute(sb0 + 2, 0)
            fire_gather(0, sg0)

        drain_gather(sg1)
        extract_write(sb0 + 1, 1)


def _conv_body(rule, base, chans_in, chans_out, stage_chan, ridx, acc_buf,
               wv, subsz, s0, s1):
    nsub = CPW // subsz
    batch = 27 * subsz

    def fire(sc, boff, sem):
        for k in range(27):
            pltpu.async_copy(
                rule.at[pl.ds(k * NPAD + base + sc * subsz, subsz)],
                ridx.at[pl.ds(boff + k * subsz, subsz)], sem)

    def drain(sem):
        for k in range(27):
            pltpu.make_async_copy(rule.at[pl.ds(0, subsz)],
                                  ridx.at[pl.ds(0, subsz)], sem).wait()

    for c in range(chans_in):
        ch_ref = stage_chan(c)
        fire(0, 0, s0)

        @pl.loop(0, nsub // 2)
        def _pipe(i):
            sc0 = i * 2
            fire(sc0 + 1, batch, s1)
            drain(s0)
            _conv_compute(c, sc0, 0, ch_ref, ridx, acc_buf, wv, subsz,
                          chans_in, chans_out)

            @pl.when(sc0 + 2 < nsub)
            def _next():
                fire(sc0 + 2, 0, s0)

            drain(s1)
            _conv_compute(c, sc0 + 1, batch, ch_ref, ridx, acc_buf, wv,
                          subsz, chans_in, chans_out)


def _conv_compute(c, sc, boff, ch_ref, ridx, acc_buf, wv, subsz, chans_in,
                  chans_out):
    @pl.loop(0, subsz // 32)
    def _grp(gg):
        for half in (0, 1):
            g = gg * 2 + half
            o = sc * subsz + g * 16
            if c == 0:
                accs = [None] * chans_out
            else:
                accs = [acc_buf[pl.ds(j * CPW + o, 16)]
                        for j in range(chans_out)]
            for k in range(27):
                idx = ridx[pl.ds(boff + k * subsz + g * 16, 16)]
                v = plsc.load_gather(ch_ref, [idx])
                for j in range(chans_out):
                    wvec = wv[pl.ds((k * chans_in + c) * chans_out + j,
                                    16)][0]
                    t = wvec * v
                    accs[j] = t if accs[j] is None else accs[j] + t
            for j in range(chans_out):
                acc_buf[pl.ds(j * CPW + o, 16)] = accs[j]


@functools.partial(
    pl.kernel,
    out_type=jax.ShapeDtypeStruct((3 * CH,), jnp.float32),
    mesh=_mesh2,
    compiler_params=_params,
    scratch_types=[
        pltpu.VMEM((CH,), jnp.float32),
        pltpu.VMEM((2 * 27 * 320,), jnp.int32),
        pltpu.VMEM((3 * CPW,), jnp.float32),
        pltpu.VMEM((184,), jnp.float32),
        pltpu.VMEM((16,), jnp.float32),
        pltpu.SemaphoreType.DMA,
        pltpu.SemaphoreType.DMA,
    ],
)
def _k3_conv1(rule, fcs, w1f, h, fc, ridx, hacc, w1v, z16, s0, s1):
    w = _wid()
    base = w * CPW
    pltpu.sync_copy(w1f, w1v)

    def stage_chan(c):
        pltpu.sync_copy(fcs.at[pl.ds(c * CH, CH)], fc)
        return fc

    _conv_body(rule, base, 2, 3, stage_chan, ridx, hacc, w1v, 320, s0, s1)

    @pl.when(w == 0)
    def _zero_head():
        z16[...] = jnp.zeros((16,), jnp.float32)
        for j in range(3):
            pltpu.sync_copy(z16, h.at[pl.ds(j * CH, 16)])
            pltpu.sync_copy(z16, h.at[pl.ds(j * CH + CH - 16, 16)])
    for j in range(3):
        pltpu.sync_copy(hacc.at[pl.ds(j * CPW, CPW)],
                        h.at[pl.ds(j * CH + 8 + base, CPW)])


@functools.partial(
    pl.kernel,
    out_type=jax.ShapeDtypeStruct((4 * NPAD,), jnp.float32),
    mesh=_mesh2,
    compiler_params=_params,
    scratch_types=[
        pltpu.VMEM((CH,), jnp.float32),
        pltpu.VMEM((2 * 27 * 160,), jnp.int32),
        pltpu.VMEM((4 * CPW,), jnp.float32),
        pltpu.VMEM((344,), jnp.float32),
        pltpu.SemaphoreType.DMA,
        pltpu.SemaphoreType.DMA,
    ],
)
def _k4_conv2(rule, h, w2f, out, hch, ridx, oacc, w2v, s0, s1):
    w = _wid()
    base = w * CPW
    pltpu.sync_copy(w2f, w2v)

    def stage_chan(c):
        pltpu.sync_copy(h.at[pl.ds(c * CH, CH)], hch)
        return hch

    _conv_body(rule, base, 3, 4, stage_chan, ridx, oacc, w2v, 160, s0, s1)

    for j in range(4):
        pltpu.sync_copy(oacc.at[pl.ds(j * CPW, CPW)],
                        out.at[pl.ds(j * NPAD + base, CPW)])


def kernel(features, coords, W1, W2):
    z = coords[:, 0].astype(jnp.int32)
    y = coords[:, 1].astype(jnp.int32)
    x = coords[:, 2].astype(jnp.int32)
    line = z * GRID_D + y
    lin_a = line * GRID_D + x
    dump = jnp.arange(PADPTS, dtype=jnp.int32)
    lin1 = jnp.concatenate([lin_a, DUMPA + dump])
    vals1 = jnp.arange(NPAD, dtype=jnp.int32) + 8

    pad1 = jnp.zeros((NPAD - N,), jnp.int32)
    coords1 = jnp.concatenate([z, pad1, y, pad1, x, pad1])

    fcs = jnp.zeros((2, CH), jnp.float32).at[:, 8:8 + N].set(
        features.T).reshape(-1)
    w1f = jnp.pad(W1.reshape(-1), (0, 184 - 162))
    w2f = jnp.pad(W2.reshape(-1), (0, 344 - 324))

    ga = _k1a_scatter(lin1, vals1)
    grid = _k1b_mirror(ga)
    rule = _k2_rule(grid.reshape(TROWS, 16), coords1)
    h = _k3_conv1(rule, fcs, w1f)
    outT = _k4_conv2(rule, h, w2f)
    return outT.reshape(4, NPAD).T[:N]
